# Initial kernel scaffold; baseline (speedup 1.0000x reference)
import jax
import jax.numpy as jnp
from jax.experimental import pallas as pl


def kernel(input, embedding):
    raise NotImplementedError("write your pallas kernel here")


# speedup vs baseline: 6.3082x; 6.3082x over previous
import functools

import jax
import jax.numpy as jnp
from jax import lax
from jax.experimental import pallas as pl
from jax.experimental.pallas import tpu as pltpu
from jax.experimental.pallas import tpu_sc as plsc

NUM_CORES = 2
NUM_SUBCORES = 16
NUM_WORKERS = NUM_CORES * NUM_SUBCORES
CHUNK = 128


def _build_kernel(B, D, n_chunks):
    b_per_w = n_chunks * CHUNK
    mesh = plsc.VectorSubcoreMesh(core_axis_name="c", subcore_axis_name="s")

    @functools.partial(
        pl.kernel,
        mesh=mesh,
        out_type=jax.ShapeDtypeStruct((B, D), jnp.float32),
        scratch_types=[
            pltpu.VMEM((n_chunks, CHUNK), jnp.int32),
            pltpu.VMEM((CHUNK, D), jnp.float32),
            pltpu.SemaphoreType.DMA,
        ],
    )
    def k(table_hbm, idx_hbm, out_hbm, idx_v, rows_v, sem):
        wid = lax.axis_index("s") * NUM_CORES + lax.axis_index("c")
        base = wid * b_per_w
        pltpu.sync_copy(idx_hbm.at[pl.ds(wid * n_chunks, n_chunks)], idx_v)

        def body(i, carry):
            pltpu.async_copy(table_hbm.at[idx_v.at[i]], rows_v, sem).wait()
            pltpu.sync_copy(rows_v, out_hbm.at[pl.ds(base + i * CHUNK, CHUNK)])
            return carry

        lax.fori_loop(0, n_chunks, body, 0)

    return k


def kernel(input, embedding):
    D = embedding.shape[1]
    B = input.size
    idx = input.reshape(-1).astype(jnp.int32)
    n_chunks = B // (NUM_WORKERS * CHUNK)
    idx2d = idx.reshape(NUM_WORKERS * n_chunks, CHUNK)
    out = _build_kernel(B, D, n_chunks)(embedding, idx2d)
    return out.reshape(input.shape + (D,))


# speedup vs baseline: 9.1031x; 1.4430x over previous
import functools

import jax
import jax.numpy as jnp
from jax import lax
from jax.experimental import pallas as pl
from jax.experimental.pallas import tpu as pltpu
from jax.experimental.pallas import tpu_sc as plsc

NUM_CORES = 2
NUM_SUBCORES = 16
NUM_WORKERS = NUM_CORES * NUM_SUBCORES
CHUNK = 128


NBUF = 4


def _build_kernel(B, D, n_chunks):
    b_per_w = n_chunks * CHUNK
    n_groups = n_chunks // NBUF
    mesh = plsc.VectorSubcoreMesh(core_axis_name="c", subcore_axis_name="s")

    @functools.partial(
        pl.kernel,
        mesh=mesh,
        out_type=jax.ShapeDtypeStruct((B, D), jnp.float32),
        scratch_types=[
            pltpu.VMEM((n_chunks, CHUNK), jnp.int32),
        ]
        + [pltpu.VMEM((CHUNK, D), jnp.float32) for _ in range(NBUF)]
        + [pltpu.SemaphoreType.DMA for _ in range(2 * NBUF)],
    )
    def k(table_hbm, idx_hbm, out_hbm, idx_v, *scratch):
        rows = scratch[:NBUF]
        gsem = scratch[NBUF:2 * NBUF]
        ssem = scratch[2 * NBUF:3 * NBUF]
        wid = lax.axis_index("s") * NUM_CORES + lax.axis_index("c")
        base = wid * b_per_w
        pltpu.sync_copy(idx_hbm.at[pl.ds(wid * n_chunks, n_chunks)], idx_v)

        def gather(i, b):
            pltpu.async_copy(table_hbm.at[idx_v.at[i]], rows[b], gsem[b])

        def wait_gather(i, b):
            pltpu.make_async_copy(table_hbm.at[idx_v.at[i]], rows[b],
                                  gsem[b]).wait()

        def scatter(i, b):
            pltpu.async_copy(
                rows[b], out_hbm.at[pl.ds(base + i * CHUNK, CHUNK)], ssem[b])

        def wait_scatter(i, b):
            pltpu.make_async_copy(
                rows[b], out_hbm.at[pl.ds(base + i * CHUNK, CHUNK)],
                ssem[b]).wait()

        for b in range(NBUF):
            gather(b, b)

        def group(g, carry):
            i0 = g * NBUF
            for b in range(NBUF):
                wait_gather(i0 + b, b)
                scatter(i0 + b, b)
            for b in range(NBUF):
                wait_scatter(i0 + b, b)
                gather(i0 + NBUF + b, b)
            return carry

        lax.fori_loop(0, n_groups - 1, group, 0)

        i0 = (n_groups - 1) * NBUF
        for b in range(NBUF):
            wait_gather(i0 + b, b)
            scatter(i0 + b, b)
        for b in range(NBUF):
            wait_scatter(i0 + b, b)

    return k


def kernel(input, embedding):
    D = embedding.shape[1]
    B = input.size
    idx = input.reshape(-1).astype(jnp.int32)
    n_chunks = B // (NUM_WORKERS * CHUNK)
    idx2d = idx.reshape(NUM_WORKERS * n_chunks, CHUNK)
    out = _build_kernel(B, D, n_chunks)(embedding, idx2d)
    return out.reshape(input.shape + (D,))
